# baseline (speedup 1.0000x reference)
import jax
import jax.numpy as jnp
from jax.experimental import pallas as pl
from jax.experimental.pallas import tpu as pltpu

_BS, _T, _V, _F = 16, 32, 10, 2048
_LUT = 16
_GB = 2


def _fused_kernel(s_ref, w_ref, x_ref, ein_ref, eout_ref,
                  ab_ref, o_ref, deg_ref):
    step = pl.program_id(0)

    @pl.when(step == 0)
    def _small():
        s = s_ref[...]
        dist = jnp.minimum(s, s_ref[...].transpose(1, 0, 2, 3))

        deg_ref[...] = jnp.sum(dist, axis=0)

        for k in range(_V):
            m = jnp.maximum(dist[:, k:k + 1, :, :], dist[k:k + 1, :, :, :])
            dist = dist * m

        cnt = jnp.sum(dist, axis=1)
        ab_ref[...] = ((_V - cnt)[:, None, :, :] * w_ref[0:1]
                       + cnt[:, None, :, :] * w_ref[1:2]
                       + float(_V) * w_ref[2:3])

    lut = ein_ref[...] + eout_ref[...]
    ci = jax.lax.broadcasted_iota(jnp.int32, (_LUT, 1), 0).astype(jnp.float32)
    for g in range(_GB):
        bb = step * _GB + g
        for v in range(_V):
            degv = deg_ref[v, pl.ds(bb, 1), :]
            oht = (ci == degv).astype(jnp.float32)
            rows = jax.lax.dot_general(
                oht, lut, (((0,), (0,)), ((), ())),
                preferred_element_type=jnp.float32)
            o_ref[g, v] = x_ref[g, v] + rows


def kernel(end_output, S, emb_in, emb_out, emb3, emb4, mul, bias, means, stds):
    s4 = jnp.transpose(S, (2, 3, 0, 1))
    x4 = jnp.transpose(end_output, (0, 2, 1, 3))

    wpack = jnp.concatenate(
        [emb3[0:2, :], emb4[0:1, :],
         jnp.zeros((_V - 3, _V), jnp.float32)], axis=0).reshape(_V, _V, 1, 1)

    ab4, o4 = pl.pallas_call(
        _fused_kernel,
        grid=(_BS // _GB,),
        in_specs=[pl.BlockSpec((_V, _V, _BS, _T), lambda i: (0, 0, 0, 0)),
                  pl.BlockSpec((_V, _V, 1, 1), lambda i: (0, 0, 0, 0)),
                  pl.BlockSpec((_GB, _V, _T, _F), lambda i: (i, 0, 0, 0)),
                  pl.BlockSpec((_LUT, _F), lambda i: (0, 0)),
                  pl.BlockSpec((_LUT, _F), lambda i: (0, 0))],
        out_specs=(pl.BlockSpec((_V, _V, _BS, _T), lambda i: (0, 0, 0, 0)),
                   pl.BlockSpec((_GB, _V, _T, _F), lambda i: (i, 0, 0, 0))),
        out_shape=(jax.ShapeDtypeStruct((_V, _V, _BS, _T), jnp.float32),
                   jax.ShapeDtypeStruct((_BS, _V, _T, _F), jnp.float32)),
        scratch_shapes=[pltpu.VMEM((_V, _BS, _T), jnp.float32)],
    )(s4, wpack, x4, emb_in, emb_out)

    return (jnp.transpose(o4, (0, 2, 1, 3)),
            jnp.transpose(ab4, (2, 3, 0, 1)))


# speedup vs baseline: 1.0625x; 1.0625x over previous
import jax
import jax.numpy as jnp
from jax.experimental import pallas as pl
from jax.experimental.pallas import tpu as pltpu

_BS, _T, _V, _F = 16, 32, 10, 2048
_LUT = 16
_GB = 4


def _fused_kernel(s_ref, w_ref, x_ref, ein_ref, eout_ref,
                  ab_ref, o_ref, deg_ref):
    step = pl.program_id(0)

    @pl.when(step == 0)
    def _small():
        s = s_ref[...]
        dist = jnp.minimum(s, s_ref[...].transpose(1, 0, 2, 3))

        deg_ref[...] = jnp.sum(dist, axis=0)

        for k in range(_V):
            m = jnp.maximum(dist[:, k:k + 1, :, :], dist[k:k + 1, :, :, :])
            dist = dist * m

        cnt = jnp.sum(dist, axis=1)
        ab_ref[...] = ((_V - cnt)[:, None, :, :] * w_ref[0:1]
                       + cnt[:, None, :, :] * w_ref[1:2]
                       + float(_V) * w_ref[2:3])

    lut = ein_ref[...] + eout_ref[...]
    ci = jax.lax.broadcasted_iota(jnp.int32, (_LUT, 1), 0).astype(jnp.float32)
    for g in range(_GB):
        bb = step * _GB + g
        for v in range(_V):
            degv = deg_ref[v, pl.ds(bb, 1), :]
            oht = (ci == degv).astype(jnp.float32)
            rows = jax.lax.dot_general(
                oht, lut, (((0,), (0,)), ((), ())),
                preferred_element_type=jnp.float32)
            o_ref[g, v] = x_ref[g, v] + rows


def kernel(end_output, S, emb_in, emb_out, emb3, emb4, mul, bias, means, stds):
    s4 = jnp.transpose(S, (2, 3, 0, 1))
    x4 = jnp.transpose(end_output, (0, 2, 1, 3))

    wpack = jnp.concatenate(
        [emb3[0:2, :], emb4[0:1, :],
         jnp.zeros((_V - 3, _V), jnp.float32)], axis=0).reshape(_V, _V, 1, 1)

    ab4, o4 = pl.pallas_call(
        _fused_kernel,
        grid=(_BS // _GB,),
        in_specs=[pl.BlockSpec((_V, _V, _BS, _T), lambda i: (0, 0, 0, 0)),
                  pl.BlockSpec((_V, _V, 1, 1), lambda i: (0, 0, 0, 0)),
                  pl.BlockSpec((_GB, _V, _T, _F), lambda i: (i, 0, 0, 0)),
                  pl.BlockSpec((_LUT, _F), lambda i: (0, 0)),
                  pl.BlockSpec((_LUT, _F), lambda i: (0, 0))],
        out_specs=(pl.BlockSpec((_V, _V, _BS, _T), lambda i: (0, 0, 0, 0)),
                   pl.BlockSpec((_GB, _V, _T, _F), lambda i: (i, 0, 0, 0))),
        out_shape=(jax.ShapeDtypeStruct((_V, _V, _BS, _T), jnp.float32),
                   jax.ShapeDtypeStruct((_BS, _V, _T, _F), jnp.float32)),
        scratch_shapes=[pltpu.VMEM((_V, _BS, _T), jnp.float32)],
    )(s4, wpack, x4, emb_in, emb_out)

    return (jnp.transpose(o4, (0, 2, 1, 3)),
            jnp.transpose(ab4, (2, 3, 0, 1)))
